# baseline (speedup 1.0000x reference)
import functools

import jax
import jax.numpy as jnp
from jax import lax
from jax.experimental import pallas as pl
from jax.experimental.pallas import tpu as pltpu
from jax.experimental.pallas import tpu_sc as plsc

KNN = 32
EPS = 1e-5


def _knn_body(pq_ref, bq_ref, pc_ref, bc_ref, ball_ref, out_ref, *, T, Q):
    pq = pq_ref[...]
    qx = pq[0:1, :]
    qy = pq[1:2, :]
    qz = pq[2:3, :]
    bq = bq_ref[...]
    b_all = ball_ref[...]
    min_b = jnp.min(bq)
    max_b = jnp.max(bq)
    lo = jnp.sum((b_all < min_b).astype(jnp.int32))
    hi = jnp.sum((b_all <= max_b).astype(jnp.int32))
    lo8 = (lo // 8) * 8
    nt = (hi - lo8 + T - 1) // T
    inf = jnp.float32(jnp.inf)

    big = jnp.int32(2 ** 30)
    row32 = lax.broadcasted_iota(jnp.int32, (KNN, Q), 0)

    def tile_body(t, carry):
        run_d, run_i = carry
        off = lo8 + t * T
        pc = pc_ref[pl.ds(off, T), :]
        cb = bc_ref[pl.ds(off, T), :]
        dx = pc[:, 0:1] - qx
        dy = pc[:, 1:2] - qy
        dz = pc[:, 2:3] - qz
        d2 = (dx * dx + dy * dy) + dz * dz
        keep = (cb == bq) & (d2 < run_d[KNN - 1:KNN, :])
        d2 = jnp.where(keep, d2, inf)
        gi = lax.broadcasted_iota(jnp.int32, (T, Q), 0) + off

        def cond(st):
            k, mrow, ws_d, nd, ni = st
            return (k < KNN) & (jnp.min(mrow) < inf)

        def body(st):
            k, mrow, ws_d, nd, ni = st
            imin = jnp.min(jnp.where(ws_d == mrow, gi, big), axis=0,
                           keepdims=True)
            upd = (row32 == (KNN - 1 - k)) & (mrow < inf)
            nd = jnp.where(upd, mrow, nd)
            ni = jnp.where(upd, imin, ni)
            ws_d = jnp.where(gi == imin, inf, ws_d)
            mrow = jnp.min(ws_d, axis=0, keepdims=True)
            return (k + 1, mrow, ws_d, nd, ni)

        mrow0 = jnp.min(d2, axis=0, keepdims=True)
        nd0 = jnp.full((KNN, Q), inf, jnp.float32)
        ni0 = jnp.full((KNN, Q), -1, jnp.int32)
        st = (jnp.int32(0), mrow0, d2, nd0, ni0)
        _, _, _, nd, ni = lax.while_loop(cond, body, st)

        zd = jnp.concatenate([run_d, nd], axis=0)
        zi = jnp.concatenate([run_i, ni], axis=0)
        for d in (KNN, KNN // 2, KNN // 4, KNN // 8, 2, 1):
            zd3 = zd.reshape(2 * KNN // (2 * d), 2 * d, Q)
            zi3 = zi.reshape(2 * KNN // (2 * d), 2 * d, Q)
            ad, bd = zd3[:, :d, :], zd3[:, d:, :]
            ai, bi_ = zi3[:, :d, :], zi3[:, d:, :]
            a_wins = (ad < bd) | ((ad == bd) & (ai < bi_))
            lo_d = jnp.where(a_wins, ad, bd)
            lo_i = jnp.where(a_wins, ai, bi_)
            hi_d = jnp.where(a_wins, bd, ad)
            hi_i = jnp.where(a_wins, bi_, ai)
            zd = jnp.concatenate([lo_d, hi_d], axis=1).reshape(2 * KNN, Q)
            zi = jnp.concatenate([lo_i, hi_i], axis=1).reshape(2 * KNN, Q)
        return zd[:KNN], zi[:KNN]

    run_d = jnp.full((KNN, Q), inf, jnp.float32)
    run_i = lax.broadcasted_iota(jnp.int32, (KNN, Q), 0) - KNN
    _, run_i = lax.fori_loop(0, nt, tile_body, (run_d, run_i))
    out_ref[...] = run_i


def _knn(p_pad, b_col, p_t, b_row, npad, Q=512, T=512):
    grid = npad // Q
    npad2 = npad + T
    return pl.pallas_call(
        functools.partial(_knn_body, T=T, Q=Q),
        grid=(grid,),
        in_specs=[
            pl.BlockSpec((3, Q), lambda i: (0, i)),
            pl.BlockSpec((1, Q), lambda i: (0, i)),
            pl.BlockSpec((npad2, 3), lambda i: (0, 0)),
            pl.BlockSpec((npad2, 1), lambda i: (0, 0)),
            pl.BlockSpec((1, npad), lambda i: (0, 0)),
        ],
        out_specs=pl.BlockSpec((KNN, Q), lambda i: (0, i)),
        out_shape=jax.ShapeDtypeStruct((KNN, npad), jnp.int32),
    )(p_t, b_row, p_pad, b_col, b_row)


def _uz_body(p_ref, x_ref, w1a_ref, w1b_ref, b1_ref, u_ref, z_ref):
    p = p_ref[...]
    w1a = w1a_ref[...]
    t = (p[:, 0:1] * w1a[0:1, :] + p[:, 1:2] * w1a[1:2, :]
         + p[:, 2:3] * w1a[2:3, :])
    u_ref[...] = t + jnp.dot(x_ref[...], w1b_ref[...],
                             preferred_element_type=jnp.float32)
    z_ref[...] = t - b1_ref[...]


def _uz(p, x, w1a, w1b, b1, n, c, BQ=400):
    grid = n // BQ
    return pl.pallas_call(
        _uz_body,
        grid=(grid,),
        in_specs=[
            pl.BlockSpec((BQ, 3), lambda i: (i, 0)),
            pl.BlockSpec((BQ, c), lambda i: (i, 0)),
            pl.BlockSpec((3, 128), lambda i: (0, 0)),
            pl.BlockSpec((c, 128), lambda i: (0, 0)),
            pl.BlockSpec((1, 128), lambda i: (0, 0)),
        ],
        out_specs=[
            pl.BlockSpec((BQ, 128), lambda i: (i, 0)),
            pl.BlockSpec((BQ, 128), lambda i: (i, 0)),
        ],
        out_shape=[
            jax.ShapeDtypeStruct((n, 128), jnp.float32),
            jax.ShapeDtypeStruct((n, 128), jnp.float32),
        ],
    )(p, x, w1a, w1b, b1)


def _sc_gather(u, col_flat, E, D):
    info = plsc.get_sparse_core_info()
    nw = info.num_cores * info.num_subcores
    per_w = E // nw
    CH = 400
    n_ch = per_w // CH
    mesh = plsc.VectorSubcoreMesh(core_axis_name="c", subcore_axis_name="s")

    @functools.partial(
        pl.kernel,
        mesh=mesh,
        out_type=jax.ShapeDtypeStruct((E, D), jnp.float32),
        scratch_types=[
            pltpu.VMEM((CH,), jnp.int32),
            pltpu.VMEM((CH, D), jnp.float32),
            pltpu.SemaphoreType.DMA,
        ],
    )
    def gk(u_hbm, idx_hbm, out_hbm, idx_v, rows_v, sem):
        wid = lax.axis_index("s") * info.num_cores + lax.axis_index("c")
        base = wid * per_w

        def body(i, carry):
            o = base + i * CH
            pltpu.sync_copy(idx_hbm.at[pl.ds(o, CH)], idx_v)
            pltpu.async_copy(u_hbm.at[idx_v], rows_v, sem).wait()
            pltpu.sync_copy(rows_v, out_hbm.at[pl.ds(o, CH)])
            return carry

        lax.fori_loop(0, n_ch, body, 0)

    return gk(u, col_flat)


def _stats1_body(g_ref, z_ref, s_ref, ss_ref):
    bq = g_ref.shape[0]
    h = g_ref[...] - z_ref[...][:, None, :]
    h = h.reshape(bq * KNN, 128)

    @pl.when(pl.program_id(0) == 0)
    def _():
        s_ref[...] = jnp.zeros_like(s_ref)
        ss_ref[...] = jnp.zeros_like(ss_ref)

    s_ref[...] += jnp.sum(h, axis=0)[None, :]
    ss_ref[...] += jnp.sum(h * h, axis=0)[None, :]


def _bn_coeffs(s_ref, ss_ref, g_ref, be_ref, n_edges):
    mu = s_ref[...] / n_edges
    var = ss_ref[...] / n_edges - mu * mu
    sc = g_ref[...] * lax.rsqrt(var + EPS)
    return sc, be_ref[...] - mu * sc


def _stats2_body(g_ref, z_ref, s1_ref, ss1_ref, g1_ref, be1_ref,
                 w2_ref, b2_ref, s_ref, ss_ref, *, n_edges):
    bq = g_ref.shape[0]
    sc1, sh1 = _bn_coeffs(s1_ref, ss1_ref, g1_ref, be1_ref, n_edges)
    h = g_ref[...] - z_ref[...][:, None, :]
    h = h.reshape(bq * KNN, 128)
    a = jnp.maximum(h * sc1 + sh1, 0.0)
    h2 = jnp.dot(a, w2_ref[...], preferred_element_type=jnp.float32)
    h2 = h2 + b2_ref[...]

    @pl.when(pl.program_id(0) == 0)
    def _():
        s_ref[...] = jnp.zeros_like(s_ref)
        ss_ref[...] = jnp.zeros_like(ss_ref)

    s_ref[...] += jnp.sum(h2, axis=0)[None, :]
    ss_ref[...] += jnp.sum(h2 * h2, axis=0)[None, :]


def _final_body(g_ref, z_ref, s1_ref, ss1_ref, g1_ref, be1_ref,
                w2_ref, b2_ref, s2_ref, ss2_ref, g2_ref, be2_ref,
                out_ref, *, n_edges):
    bq = g_ref.shape[0]
    sc1, sh1 = _bn_coeffs(s1_ref, ss1_ref, g1_ref, be1_ref, n_edges)
    sc2, sh2 = _bn_coeffs(s2_ref, ss2_ref, g2_ref, be2_ref, n_edges)
    h = g_ref[...] - z_ref[...][:, None, :]
    h = h.reshape(bq * KNN, 128)
    a = jnp.maximum(h * sc1 + sh1, 0.0)
    h2 = jnp.dot(a, w2_ref[...], preferred_element_type=jnp.float32)
    a2 = jnp.maximum((h2 + b2_ref[...]) * sc2 + sh2, 0.0)
    out_ref[...] = jnp.max(a2.reshape(bq, KNN, 128), axis=1)


def _run_passes(g3, z, g1, be1, w2, b2, g2, be2, n, BQ=200):
    grid = n // BQ
    n_edges = float(n * KNN)
    vec = pl.BlockSpec((1, 128), lambda i: (0, 0))
    g_spec = pl.BlockSpec((BQ, KNN, 128), lambda i: (i, 0, 0))
    z_spec = pl.BlockSpec((BQ, 128), lambda i: (i, 0))
    w_spec = pl.BlockSpec((128, 128), lambda i: (0, 0))
    acc = jax.ShapeDtypeStruct((1, 128), jnp.float32)

    s1, ss1 = pl.pallas_call(
        _stats1_body,
        grid=(grid,),
        in_specs=[g_spec, z_spec],
        out_specs=[vec, vec],
        out_shape=[acc, acc],
    )(g3, z)

    s2, ss2 = pl.pallas_call(
        functools.partial(_stats2_body, n_edges=n_edges),
        grid=(grid,),
        in_specs=[g_spec, z_spec, vec, vec, vec, vec, w_spec, vec],
        out_specs=[vec, vec],
        out_shape=[acc, acc],
    )(g3, z, s1, ss1, g1, be1, w2, b2)

    return pl.pallas_call(
        functools.partial(_final_body, n_edges=n_edges),
        grid=(grid,),
        in_specs=[g_spec, z_spec, vec, vec, vec, vec, w_spec, vec,
                  vec, vec, vec, vec],
        out_specs=pl.BlockSpec((BQ, 128), lambda i: (i, 0)),
        out_shape=jax.ShapeDtypeStruct((n, 128), jnp.float32),
    )(g3, z, s1, ss1, g1, be1, w2, b2, s2, ss2, g2, be2)


def kernel(p, x, b, W1, bias1, g1, be1, W2, bias2, g2, be2):
    n, c = x.shape
    bi = b.astype(jnp.int32)
    npad = ((n + 511) // 512) * 512
    pad = npad - n

    p_pad = jnp.pad(p, ((0, pad + 512), (0, 0)))
    b_pad = jnp.pad(bi, (0, pad + 512), constant_values=127)
    col = _knn(p_pad, b_pad[:, None], p_pad[:npad].T, b_pad[None, :npad],
               npad)[:, :n].T

    row_vec = lambda v: v[None, :]
    u, z = _uz(p, x, W1[:3], W1[3:], row_vec(bias1), n, c)

    e = n * KNN
    g = _sc_gather(u, col.reshape(e), e, 128)
    g3 = g.reshape(n, KNN, 128)

    x_agg = _run_passes(g3, z, row_vec(g1), row_vec(be1), W2,
                        row_vec(bias2), row_vec(g2), row_vec(be2), n)
    return (p, x_agg, b)


# speedup vs baseline: 1.2620x; 1.2620x over previous
import functools

import jax
import jax.numpy as jnp
from jax import lax
from jax.experimental import pallas as pl
from jax.experimental.pallas import tpu as pltpu
from jax.experimental.pallas import tpu_sc as plsc

KNN = 32
EPS = 1e-5


def _knn_body(pq_ref, bq_ref, pc_ref, bc_ref, ball_ref, out_ref, *, T, Q):
    pq = pq_ref[...]
    qx = pq[0:1, :]
    qy = pq[1:2, :]
    qz = pq[2:3, :]
    bq = bq_ref[...]
    b_all = ball_ref[...]
    min_b = jnp.min(bq)
    max_b = jnp.max(bq)
    lo = jnp.sum((b_all < min_b).astype(jnp.int32))
    hi = jnp.sum((b_all <= max_b).astype(jnp.int32))
    lo8 = (lo // 8) * 8
    nt = (hi - lo8 + T - 1) // T
    inf = jnp.float32(jnp.inf)

    def tile_body(t, carry):
        run_d, run_i = carry
        off = lo8 + t * T
        pc = pc_ref[pl.ds(off, T), :]
        cb = bc_ref[pl.ds(off, T), :]
        dx = pc[:, 0:1] - qx
        dy = pc[:, 1:2] - qy
        dz = pc[:, 2:3] - qz
        d2 = (dx * dx + dy * dy) + dz * dz
        d2 = jnp.where(cb == bq, d2, inf)
        gi = lax.broadcasted_iota(jnp.int32, (T, Q), 0) + off
        ws_d = jnp.concatenate([run_d, d2], axis=0)
        ws_i = jnp.concatenate([run_i, gi], axis=0)
        rows_d = []
        rows_i = []
        big = jnp.int32(2 ** 30)
        for _ in range(KNN):
            m = jnp.min(ws_d, axis=0, keepdims=True)
            imin = jnp.min(jnp.where(ws_d == m, ws_i, big), axis=0,
                           keepdims=True)
            rows_d.append(m)
            rows_i.append(imin)
            ws_d = jnp.where(ws_i == imin, inf, ws_d)
        return (jnp.concatenate(rows_d, axis=0),
                jnp.concatenate(rows_i, axis=0))

    run_d = jnp.full((KNN, Q), inf, jnp.float32)
    run_i = lax.broadcasted_iota(jnp.int32, (KNN, Q), 0) - KNN
    _, run_i = lax.fori_loop(0, nt, tile_body, (run_d, run_i))
    out_ref[...] = run_i


def _knn(p_pad, b_col, p_t, b_row, npad, Q=256, T=512):
    grid = npad // Q
    npad2 = npad + T
    return pl.pallas_call(
        functools.partial(_knn_body, T=T, Q=Q),
        grid=(grid,),
        in_specs=[
            pl.BlockSpec((3, Q), lambda i: (0, i)),
            pl.BlockSpec((1, Q), lambda i: (0, i)),
            pl.BlockSpec((npad2, 3), lambda i: (0, 0)),
            pl.BlockSpec((npad2, 1), lambda i: (0, 0)),
            pl.BlockSpec((1, npad), lambda i: (0, 0)),
        ],
        out_specs=pl.BlockSpec((KNN, Q), lambda i: (0, i)),
        out_shape=jax.ShapeDtypeStruct((KNN, npad), jnp.int32),
    )(p_t, b_row, p_pad, b_col, b_row)


def _uz_body(p_ref, x_ref, w1a_ref, w1b_ref, b1_ref, u_ref, z_ref):
    p = p_ref[...]
    w1a = w1a_ref[...]
    t = (p[:, 0:1] * w1a[0:1, :] + p[:, 1:2] * w1a[1:2, :]
         + p[:, 2:3] * w1a[2:3, :])
    u_ref[...] = t + jnp.dot(x_ref[...], w1b_ref[...],
                             preferred_element_type=jnp.float32)
    z_ref[...] = t - b1_ref[...]


def _uz(p, x, w1a, w1b, b1, n, c, BQ=400):
    grid = n // BQ
    return pl.pallas_call(
        _uz_body,
        grid=(grid,),
        in_specs=[
            pl.BlockSpec((BQ, 3), lambda i: (i, 0)),
            pl.BlockSpec((BQ, c), lambda i: (i, 0)),
            pl.BlockSpec((3, 128), lambda i: (0, 0)),
            pl.BlockSpec((c, 128), lambda i: (0, 0)),
            pl.BlockSpec((1, 128), lambda i: (0, 0)),
        ],
        out_specs=[
            pl.BlockSpec((BQ, 128), lambda i: (i, 0)),
            pl.BlockSpec((BQ, 128), lambda i: (i, 0)),
        ],
        out_shape=[
            jax.ShapeDtypeStruct((n, 128), jnp.float32),
            jax.ShapeDtypeStruct((n, 128), jnp.float32),
        ],
    )(p, x, w1a, w1b, b1)


def _sc_gather(u, col_flat, E, D):
    info = plsc.get_sparse_core_info()
    nw = info.num_cores * info.num_subcores
    per_w = E // nw
    CH = 400
    n_ch = per_w // CH
    mesh = plsc.VectorSubcoreMesh(core_axis_name="c", subcore_axis_name="s")

    @functools.partial(
        pl.kernel,
        mesh=mesh,
        out_type=jax.ShapeDtypeStruct((E, D), jnp.float32),
        scratch_types=[
            pltpu.VMEM((CH,), jnp.int32),
            pltpu.VMEM((CH, D), jnp.float32),
            pltpu.SemaphoreType.DMA,
        ],
    )
    def gk(u_hbm, idx_hbm, out_hbm, idx_v, rows_v, sem):
        wid = lax.axis_index("s") * info.num_cores + lax.axis_index("c")
        base = wid * per_w

        def body(i, carry):
            o = base + i * CH
            pltpu.sync_copy(idx_hbm.at[pl.ds(o, CH)], idx_v)
            pltpu.async_copy(u_hbm.at[idx_v], rows_v, sem).wait()
            pltpu.sync_copy(rows_v, out_hbm.at[pl.ds(o, CH)])
            return carry

        lax.fori_loop(0, n_ch, body, 0)

    return gk(u, col_flat)


def _stats1_body(g_ref, z_ref, s_ref, ss_ref):
    bq = g_ref.shape[0]
    h = g_ref[...] - z_ref[...][:, None, :]
    h = h.reshape(bq * KNN, 128)

    @pl.when(pl.program_id(0) == 0)
    def _():
        s_ref[...] = jnp.zeros_like(s_ref)
        ss_ref[...] = jnp.zeros_like(ss_ref)

    s_ref[...] += jnp.sum(h, axis=0)[None, :]
    ss_ref[...] += jnp.sum(h * h, axis=0)[None, :]


def _bn_coeffs(s_ref, ss_ref, g_ref, be_ref, n_edges):
    mu = s_ref[...] / n_edges
    var = ss_ref[...] / n_edges - mu * mu
    sc = g_ref[...] * lax.rsqrt(var + EPS)
    return sc, be_ref[...] - mu * sc


def _stats2_body(g_ref, z_ref, s1_ref, ss1_ref, g1_ref, be1_ref,
                 w2_ref, b2_ref, s_ref, ss_ref, *, n_edges):
    bq = g_ref.shape[0]
    sc1, sh1 = _bn_coeffs(s1_ref, ss1_ref, g1_ref, be1_ref, n_edges)
    h = g_ref[...] - z_ref[...][:, None, :]
    h = h.reshape(bq * KNN, 128)
    a = jnp.maximum(h * sc1 + sh1, 0.0)
    h2 = jnp.dot(a, w2_ref[...], preferred_element_type=jnp.float32)
    h2 = h2 + b2_ref[...]

    @pl.when(pl.program_id(0) == 0)
    def _():
        s_ref[...] = jnp.zeros_like(s_ref)
        ss_ref[...] = jnp.zeros_like(ss_ref)

    s_ref[...] += jnp.sum(h2, axis=0)[None, :]
    ss_ref[...] += jnp.sum(h2 * h2, axis=0)[None, :]


def _final_body(g_ref, z_ref, s1_ref, ss1_ref, g1_ref, be1_ref,
                w2_ref, b2_ref, s2_ref, ss2_ref, g2_ref, be2_ref,
                out_ref, *, n_edges):
    bq = g_ref.shape[0]
    sc1, sh1 = _bn_coeffs(s1_ref, ss1_ref, g1_ref, be1_ref, n_edges)
    sc2, sh2 = _bn_coeffs(s2_ref, ss2_ref, g2_ref, be2_ref, n_edges)
    h = g_ref[...] - z_ref[...][:, None, :]
    h = h.reshape(bq * KNN, 128)
    a = jnp.maximum(h * sc1 + sh1, 0.0)
    h2 = jnp.dot(a, w2_ref[...], preferred_element_type=jnp.float32)
    a2 = jnp.maximum((h2 + b2_ref[...]) * sc2 + sh2, 0.0)
    out_ref[...] = jnp.max(a2.reshape(bq, KNN, 128), axis=1)


def _run_passes(g3, z, g1, be1, w2, b2, g2, be2, n, BQ=200):
    grid = n // BQ
    n_edges = float(n * KNN)
    vec = pl.BlockSpec((1, 128), lambda i: (0, 0))
    g_spec = pl.BlockSpec((BQ, KNN, 128), lambda i: (i, 0, 0))
    z_spec = pl.BlockSpec((BQ, 128), lambda i: (i, 0))
    w_spec = pl.BlockSpec((128, 128), lambda i: (0, 0))
    acc = jax.ShapeDtypeStruct((1, 128), jnp.float32)

    s1, ss1 = pl.pallas_call(
        _stats1_body,
        grid=(grid,),
        in_specs=[g_spec, z_spec],
        out_specs=[vec, vec],
        out_shape=[acc, acc],
    )(g3, z)

    s2, ss2 = pl.pallas_call(
        functools.partial(_stats2_body, n_edges=n_edges),
        grid=(grid,),
        in_specs=[g_spec, z_spec, vec, vec, vec, vec, w_spec, vec],
        out_specs=[vec, vec],
        out_shape=[acc, acc],
    )(g3, z, s1, ss1, g1, be1, w2, b2)

    return pl.pallas_call(
        functools.partial(_final_body, n_edges=n_edges),
        grid=(grid,),
        in_specs=[g_spec, z_spec, vec, vec, vec, vec, w_spec, vec,
                  vec, vec, vec, vec],
        out_specs=pl.BlockSpec((BQ, 128), lambda i: (i, 0)),
        out_shape=jax.ShapeDtypeStruct((n, 128), jnp.float32),
    )(g3, z, s1, ss1, g1, be1, w2, b2, s2, ss2, g2, be2)


def kernel(p, x, b, W1, bias1, g1, be1, W2, bias2, g2, be2):
    n, c = x.shape
    bi = b.astype(jnp.int32)
    npad = ((n + 511) // 512) * 512
    pad = npad - n

    p_pad = jnp.pad(p, ((0, pad + 512), (0, 0)))
    b_pad = jnp.pad(bi, (0, pad + 512), constant_values=127)
    col = _knn(p_pad, b_pad[:, None], p_pad[:npad].T, b_pad[None, :npad],
               npad)[:, :n].T

    row_vec = lambda v: v[None, :]
    u, z = _uz(p, x, W1[:3], W1[3:], row_vec(bias1), n, c)

    e = n * KNN
    g = _sc_gather(u, col.reshape(e), e, 128)
    g3 = g.reshape(n, KNN, 128)

    x_agg = _run_passes(g3, z, row_vec(g1), row_vec(be1), W2,
                        row_vec(bias2), row_vec(g2), row_vec(be2), n)
    return (p, x_agg, b)


# speedup vs baseline: 1.3859x; 1.0982x over previous
import functools

import jax
import jax.numpy as jnp
from jax import lax
from jax.experimental import pallas as pl
from jax.experimental.pallas import tpu as pltpu
from jax.experimental.pallas import tpu_sc as plsc

KNN = 32
EPS = 1e-5


def _knn_body(pq_ref, bq_ref, pc_ref, bc_ref, ball_ref, out_ref, *, T, Q):
    pq = pq_ref[...]
    qx = pq[0:1, :]
    qy = pq[1:2, :]
    qz = pq[2:3, :]
    bq = bq_ref[...]
    b_all = ball_ref[...]
    min_b = jnp.min(bq)
    max_b = jnp.max(bq)
    lo = jnp.sum((b_all < min_b).astype(jnp.int32))
    hi = jnp.sum((b_all <= max_b).astype(jnp.int32))
    lo8 = (lo // 8) * 8
    nt = (hi - lo8 + T - 1) // T
    inf = jnp.float32(jnp.inf)

    def tile_body(t, carry):
        run_d, run_i = carry
        off = lo8 + t * T
        pc = pc_ref[pl.ds(off, T), :]
        cb = bc_ref[pl.ds(off, T), :]
        dx = pc[:, 0:1] - qx
        dy = pc[:, 1:2] - qy
        dz = pc[:, 2:3] - qz
        d2 = (dx * dx + dy * dy) + dz * dz
        d2 = jnp.where(cb == bq, d2, inf)
        gi = lax.broadcasted_iota(jnp.int32, (T, Q), 0) + off
        ws_d = jnp.concatenate([run_d, d2], axis=0)
        ws_i = jnp.concatenate([run_i, gi], axis=0)
        rows_d = []
        rows_i = []
        big = jnp.int32(2 ** 30)
        for _ in range(KNN):
            m = jnp.min(ws_d, axis=0, keepdims=True)
            imin = jnp.min(jnp.where(ws_d == m, ws_i, big), axis=0,
                           keepdims=True)
            rows_d.append(m)
            rows_i.append(imin)
            ws_d = jnp.where(ws_i == imin, inf, ws_d)
        return (jnp.concatenate(rows_d, axis=0),
                jnp.concatenate(rows_i, axis=0))

    run_d = jnp.full((KNN, Q), inf, jnp.float32)
    run_i = lax.broadcasted_iota(jnp.int32, (KNN, Q), 0) - KNN
    _, run_i = lax.fori_loop(0, nt, tile_body, (run_d, run_i))
    out_ref[...] = run_i


def _knn(p_pad, b_col, p_t, b_row, npad, Q=512, T=512):
    grid = npad // Q
    npad2 = npad + T
    return pl.pallas_call(
        functools.partial(_knn_body, T=T, Q=Q),
        grid=(grid,),
        in_specs=[
            pl.BlockSpec((3, Q), lambda i: (0, i)),
            pl.BlockSpec((1, Q), lambda i: (0, i)),
            pl.BlockSpec((npad2, 3), lambda i: (0, 0)),
            pl.BlockSpec((npad2, 1), lambda i: (0, 0)),
            pl.BlockSpec((1, npad), lambda i: (0, 0)),
        ],
        out_specs=pl.BlockSpec((KNN, Q), lambda i: (0, i)),
        out_shape=jax.ShapeDtypeStruct((KNN, npad), jnp.int32),
    )(p_t, b_row, p_pad, b_col, b_row)


def _uz_body(p_ref, x_ref, w1a_ref, w1b_ref, b1_ref, u_ref, z_ref):
    p = p_ref[...]
    w1a = w1a_ref[...]
    t = (p[:, 0:1] * w1a[0:1, :] + p[:, 1:2] * w1a[1:2, :]
         + p[:, 2:3] * w1a[2:3, :])
    u_ref[...] = t + jnp.dot(x_ref[...], w1b_ref[...],
                             preferred_element_type=jnp.float32)
    z_ref[...] = t - b1_ref[...]


def _uz(p, x, w1a, w1b, b1, n, c, BQ=400):
    grid = n // BQ
    return pl.pallas_call(
        _uz_body,
        grid=(grid,),
        in_specs=[
            pl.BlockSpec((BQ, 3), lambda i: (i, 0)),
            pl.BlockSpec((BQ, c), lambda i: (i, 0)),
            pl.BlockSpec((3, 128), lambda i: (0, 0)),
            pl.BlockSpec((c, 128), lambda i: (0, 0)),
            pl.BlockSpec((1, 128), lambda i: (0, 0)),
        ],
        out_specs=[
            pl.BlockSpec((BQ, 128), lambda i: (i, 0)),
            pl.BlockSpec((BQ, 128), lambda i: (i, 0)),
        ],
        out_shape=[
            jax.ShapeDtypeStruct((n, 128), jnp.float32),
            jax.ShapeDtypeStruct((n, 128), jnp.float32),
        ],
    )(p, x, w1a, w1b, b1)


def _sc_gather(u, col_flat, E, D):
    info = plsc.get_sparse_core_info()
    nw = info.num_cores * info.num_subcores
    per_w = E // nw
    CH = 400
    n_ch = per_w // CH
    mesh = plsc.VectorSubcoreMesh(core_axis_name="c", subcore_axis_name="s")

    @functools.partial(
        pl.kernel,
        mesh=mesh,
        out_type=jax.ShapeDtypeStruct((E, D), jnp.float32),
        scratch_types=[
            pltpu.VMEM((CH,), jnp.int32),
            pltpu.VMEM((CH, D), jnp.float32),
            pltpu.SemaphoreType.DMA,
        ],
    )
    def gk(u_hbm, idx_hbm, out_hbm, idx_v, rows_v, sem):
        wid = lax.axis_index("s") * info.num_cores + lax.axis_index("c")
        base = wid * per_w

        def body(i, carry):
            o = base + i * CH
            pltpu.sync_copy(idx_hbm.at[pl.ds(o, CH)], idx_v)
            pltpu.async_copy(u_hbm.at[idx_v], rows_v, sem).wait()
            pltpu.sync_copy(rows_v, out_hbm.at[pl.ds(o, CH)])
            return carry

        lax.fori_loop(0, n_ch, body, 0)

    return gk(u, col_flat)


def _stats1_body(g_ref, z_ref, s_ref, ss_ref):
    bq = g_ref.shape[0]
    h = g_ref[...] - z_ref[...][:, None, :]
    h = h.reshape(bq * KNN, 128)

    @pl.when(pl.program_id(0) == 0)
    def _():
        s_ref[...] = jnp.zeros_like(s_ref)
        ss_ref[...] = jnp.zeros_like(ss_ref)

    s_ref[...] += jnp.sum(h, axis=0)[None, :]
    ss_ref[...] += jnp.sum(h * h, axis=0)[None, :]


def _bn_coeffs(s_ref, ss_ref, g_ref, be_ref, n_edges):
    mu = s_ref[...] / n_edges
    var = ss_ref[...] / n_edges - mu * mu
    sc = g_ref[...] * lax.rsqrt(var + EPS)
    return sc, be_ref[...] - mu * sc


def _stats2_body(g_ref, z_ref, s1_ref, ss1_ref, g1_ref, be1_ref,
                 w2_ref, b2_ref, s_ref, ss_ref, *, n_edges):
    bq = g_ref.shape[0]
    sc1, sh1 = _bn_coeffs(s1_ref, ss1_ref, g1_ref, be1_ref, n_edges)
    h = g_ref[...] - z_ref[...][:, None, :]
    h = h.reshape(bq * KNN, 128)
    a = jnp.maximum(h * sc1 + sh1, 0.0)
    h2 = jnp.dot(a, w2_ref[...], preferred_element_type=jnp.float32)
    h2 = h2 + b2_ref[...]

    @pl.when(pl.program_id(0) == 0)
    def _():
        s_ref[...] = jnp.zeros_like(s_ref)
        ss_ref[...] = jnp.zeros_like(ss_ref)

    s_ref[...] += jnp.sum(h2, axis=0)[None, :]
    ss_ref[...] += jnp.sum(h2 * h2, axis=0)[None, :]


def _final_body(g_ref, z_ref, s1_ref, ss1_ref, g1_ref, be1_ref,
                w2_ref, b2_ref, s2_ref, ss2_ref, g2_ref, be2_ref,
                out_ref, *, n_edges):
    bq = g_ref.shape[0]
    sc1, sh1 = _bn_coeffs(s1_ref, ss1_ref, g1_ref, be1_ref, n_edges)
    sc2, sh2 = _bn_coeffs(s2_ref, ss2_ref, g2_ref, be2_ref, n_edges)
    h = g_ref[...] - z_ref[...][:, None, :]
    h = h.reshape(bq * KNN, 128)
    a = jnp.maximum(h * sc1 + sh1, 0.0)
    h2 = jnp.dot(a, w2_ref[...], preferred_element_type=jnp.float32)
    a2 = jnp.maximum((h2 + b2_ref[...]) * sc2 + sh2, 0.0)
    out_ref[...] = jnp.max(a2.reshape(bq, KNN, 128), axis=1)


def _run_passes(g3, z, g1, be1, w2, b2, g2, be2, n, BQ=400):
    grid = n // BQ
    n_edges = float(n * KNN)
    vec = pl.BlockSpec((1, 128), lambda i: (0, 0))
    g_spec = pl.BlockSpec((BQ, KNN, 128), lambda i: (i, 0, 0))
    z_spec = pl.BlockSpec((BQ, 128), lambda i: (i, 0))
    w_spec = pl.BlockSpec((128, 128), lambda i: (0, 0))
    acc = jax.ShapeDtypeStruct((1, 128), jnp.float32)

    s1, ss1 = pl.pallas_call(
        _stats1_body,
        grid=(grid,),
        in_specs=[g_spec, z_spec],
        out_specs=[vec, vec],
        out_shape=[acc, acc],
    )(g3, z)

    s2, ss2 = pl.pallas_call(
        functools.partial(_stats2_body, n_edges=n_edges),
        grid=(grid,),
        in_specs=[g_spec, z_spec, vec, vec, vec, vec, w_spec, vec],
        out_specs=[vec, vec],
        out_shape=[acc, acc],
    )(g3, z, s1, ss1, g1, be1, w2, b2)

    return pl.pallas_call(
        functools.partial(_final_body, n_edges=n_edges),
        grid=(grid,),
        in_specs=[g_spec, z_spec, vec, vec, vec, vec, w_spec, vec,
                  vec, vec, vec, vec],
        out_specs=pl.BlockSpec((BQ, 128), lambda i: (i, 0)),
        out_shape=jax.ShapeDtypeStruct((n, 128), jnp.float32),
    )(g3, z, s1, ss1, g1, be1, w2, b2, s2, ss2, g2, be2)


def kernel(p, x, b, W1, bias1, g1, be1, W2, bias2, g2, be2):
    n, c = x.shape
    bi = b.astype(jnp.int32)
    npad = ((n + 511) // 512) * 512
    pad = npad - n

    p_pad = jnp.pad(p, ((0, pad + 512), (0, 0)))
    b_pad = jnp.pad(bi, (0, pad + 512), constant_values=127)
    col = _knn(p_pad, b_pad[:, None], p_pad[:npad].T, b_pad[None, :npad],
               npad)[:, :n].T

    row_vec = lambda v: v[None, :]
    u, z = _uz(p, x, W1[:3], W1[3:], row_vec(bias1), n, c)

    e = n * KNN
    g = _sc_gather(u, col.reshape(e), e, 128)
    g3 = g.reshape(n, KNN, 128)

    x_agg = _run_passes(g3, z, row_vec(g1), row_vec(be1), W2,
                        row_vec(bias2), row_vec(g2), row_vec(be2), n)
    return (p, x_agg, b)


# speedup vs baseline: 1.4027x; 1.0121x over previous
import functools

import jax
import jax.numpy as jnp
from jax import lax
from jax.experimental import pallas as pl
from jax.experimental.pallas import tpu as pltpu
from jax.experimental.pallas import tpu_sc as plsc

KNN = 32
EPS = 1e-5


def _knn_body(pq_ref, bq_ref, pc_ref, bc_ref, ball_ref, out_ref, *, T, Q):
    pq = pq_ref[...]
    qx = pq[0:1, :]
    qy = pq[1:2, :]
    qz = pq[2:3, :]
    bq = bq_ref[...]
    b_all = ball_ref[...]
    min_b = jnp.min(bq)
    max_b = jnp.max(bq)
    lo = jnp.sum((b_all < min_b).astype(jnp.int32))
    hi = jnp.sum((b_all <= max_b).astype(jnp.int32))
    lo8 = (lo // 8) * 8
    nt = (hi - lo8 + T - 1) // T
    inf = jnp.float32(jnp.inf)

    def tile_body(t, carry):
        run_d, run_i = carry
        off = lo8 + t * T
        pc = pc_ref[pl.ds(off, T), :]
        cb = bc_ref[pl.ds(off, T), :]
        dx = pc[:, 0:1] - qx
        dy = pc[:, 1:2] - qy
        dz = pc[:, 2:3] - qz
        d2 = (dx * dx + dy * dy) + dz * dz
        d2 = jnp.where(cb == bq, d2, inf)
        gi = lax.broadcasted_iota(jnp.int32, (T, Q), 0) + off
        ws_d = jnp.concatenate([run_d, d2], axis=0)
        ws_i = jnp.concatenate([run_i, gi], axis=0)
        rows_d = []
        rows_i = []
        big = jnp.int32(2 ** 30)
        for _ in range(KNN):
            m = jnp.min(ws_d, axis=0, keepdims=True)
            imin = jnp.min(jnp.where(ws_d == m, ws_i, big), axis=0,
                           keepdims=True)
            rows_d.append(m)
            rows_i.append(imin)
            ws_d = jnp.where(ws_i == imin, inf, ws_d)
        return (jnp.concatenate(rows_d, axis=0),
                jnp.concatenate(rows_i, axis=0))

    run_d = jnp.full((KNN, Q), inf, jnp.float32)
    run_i = lax.broadcasted_iota(jnp.int32, (KNN, Q), 0) - KNN
    _, run_i = lax.fori_loop(0, nt, tile_body, (run_d, run_i))
    out_ref[...] = run_i


def _knn(p_pad, b_col, p_t, b_row, npad, Q=512, T=512):
    grid = npad // Q
    npad2 = npad + T
    return pl.pallas_call(
        functools.partial(_knn_body, T=T, Q=Q),
        grid=(grid,),
        in_specs=[
            pl.BlockSpec((3, Q), lambda i: (0, i)),
            pl.BlockSpec((1, Q), lambda i: (0, i)),
            pl.BlockSpec((npad2, 3), lambda i: (0, 0)),
            pl.BlockSpec((npad2, 1), lambda i: (0, 0)),
            pl.BlockSpec((1, npad), lambda i: (0, 0)),
        ],
        out_specs=pl.BlockSpec((KNN, Q), lambda i: (0, i)),
        out_shape=jax.ShapeDtypeStruct((KNN, npad), jnp.int32),
    )(p_t, b_row, p_pad, b_col, b_row)


def _uz_body(p_ref, x_ref, w1a_ref, w1b_ref, b1_ref, u_ref, z_ref):
    p = p_ref[...]
    w1a = w1a_ref[...]
    t = (p[:, 0:1] * w1a[0:1, :] + p[:, 1:2] * w1a[1:2, :]
         + p[:, 2:3] * w1a[2:3, :])
    u_ref[...] = t + jnp.dot(x_ref[...], w1b_ref[...],
                             preferred_element_type=jnp.float32)
    z_ref[...] = t - b1_ref[...]


def _uz(p, x, w1a, w1b, b1, n, c, BQ=400):
    grid = n // BQ
    return pl.pallas_call(
        _uz_body,
        grid=(grid,),
        in_specs=[
            pl.BlockSpec((BQ, 3), lambda i: (i, 0)),
            pl.BlockSpec((BQ, c), lambda i: (i, 0)),
            pl.BlockSpec((3, 128), lambda i: (0, 0)),
            pl.BlockSpec((c, 128), lambda i: (0, 0)),
            pl.BlockSpec((1, 128), lambda i: (0, 0)),
        ],
        out_specs=[
            pl.BlockSpec((BQ, 128), lambda i: (i, 0)),
            pl.BlockSpec((BQ, 128), lambda i: (i, 0)),
        ],
        out_shape=[
            jax.ShapeDtypeStruct((n, 128), jnp.float32),
            jax.ShapeDtypeStruct((n, 128), jnp.float32),
        ],
    )(p, x, w1a, w1b, b1)


def _sc_gather(u, col_flat, E, D):
    info = plsc.get_sparse_core_info()
    nw = info.num_cores * info.num_subcores
    per_w = E // nw
    CH = 200
    n_pair = per_w // (2 * CH)
    mesh = plsc.VectorSubcoreMesh(core_axis_name="c", subcore_axis_name="s")

    @functools.partial(
        pl.kernel,
        mesh=mesh,
        out_type=jax.ShapeDtypeStruct((E, D), jnp.float32),
        scratch_types=[
            pltpu.VMEM((CH,), jnp.int32),
            pltpu.VMEM((CH,), jnp.int32),
            pltpu.VMEM((CH, D), jnp.float32),
            pltpu.VMEM((CH, D), jnp.float32),
            pltpu.SemaphoreType.DMA,
            pltpu.SemaphoreType.DMA,
        ],
    )
    def gk(u_hbm, idx_hbm, out_hbm, iv0, iv1, rv0, rv1, s0, s1):
        wid = lax.axis_index("s") * info.num_cores + lax.axis_index("c")
        base = wid * per_w

        pltpu.sync_copy(idx_hbm.at[pl.ds(base, CH)], iv0)
        pltpu.async_copy(u_hbm.at[iv0], rv0, s0)

        def body(g, carry):
            a = base + 2 * g * CH
            bo = a + CH
            pltpu.sync_copy(idx_hbm.at[pl.ds(bo, CH)], iv1)
            h1 = pltpu.async_copy(u_hbm.at[iv1], rv1, s1)
            pltpu.make_async_copy(u_hbm.at[iv0], rv0, s0).wait()
            pltpu.sync_copy(rv0, out_hbm.at[pl.ds(a, CH)])

            @pl.when(g < n_pair - 1)
            def _():
                nxt = bo + CH
                pltpu.sync_copy(idx_hbm.at[pl.ds(nxt, CH)], iv0)
                pltpu.async_copy(u_hbm.at[iv0], rv0, s0)

            h1.wait()
            pltpu.sync_copy(rv1, out_hbm.at[pl.ds(bo, CH)])
            return carry

        lax.fori_loop(0, n_pair, body, 0)

    return gk(u, col_flat)


def _stats1_body(g_ref, z_ref, s_ref, ss_ref):
    bq = g_ref.shape[0]
    h = g_ref[...] - z_ref[...][:, None, :]
    h = h.reshape(bq * KNN, 128)

    @pl.when(pl.program_id(0) == 0)
    def _():
        s_ref[...] = jnp.zeros_like(s_ref)
        ss_ref[...] = jnp.zeros_like(ss_ref)

    s_ref[...] += jnp.sum(h, axis=0)[None, :]
    ss_ref[...] += jnp.sum(h * h, axis=0)[None, :]


def _bn_coeffs(s_ref, ss_ref, g_ref, be_ref, n_edges):
    mu = s_ref[...] / n_edges
    var = ss_ref[...] / n_edges - mu * mu
    sc = g_ref[...] * lax.rsqrt(var + EPS)
    return sc, be_ref[...] - mu * sc


def _stats2_body(g_ref, z_ref, s1_ref, ss1_ref, g1_ref, be1_ref,
                 w2_ref, b2_ref, s_ref, ss_ref, *, n_edges):
    bq = g_ref.shape[0]
    sc1, sh1 = _bn_coeffs(s1_ref, ss1_ref, g1_ref, be1_ref, n_edges)
    h = g_ref[...] - z_ref[...][:, None, :]
    h = h.reshape(bq * KNN, 128)
    a = jnp.maximum(h * sc1 + sh1, 0.0)
    h2 = jnp.dot(a, w2_ref[...], preferred_element_type=jnp.float32)
    h2 = h2 + b2_ref[...]

    @pl.when(pl.program_id(0) == 0)
    def _():
        s_ref[...] = jnp.zeros_like(s_ref)
        ss_ref[...] = jnp.zeros_like(ss_ref)

    s_ref[...] += jnp.sum(h2, axis=0)[None, :]
    ss_ref[...] += jnp.sum(h2 * h2, axis=0)[None, :]


def _final_body(g_ref, z_ref, s1_ref, ss1_ref, g1_ref, be1_ref,
                w2_ref, b2_ref, s2_ref, ss2_ref, g2_ref, be2_ref,
                out_ref, *, n_edges):
    bq = g_ref.shape[0]
    sc1, sh1 = _bn_coeffs(s1_ref, ss1_ref, g1_ref, be1_ref, n_edges)
    sc2, sh2 = _bn_coeffs(s2_ref, ss2_ref, g2_ref, be2_ref, n_edges)
    h = g_ref[...] - z_ref[...][:, None, :]
    h = h.reshape(bq * KNN, 128)
    a = jnp.maximum(h * sc1 + sh1, 0.0)
    h2 = jnp.dot(a, w2_ref[...], preferred_element_type=jnp.float32)
    a2 = jnp.maximum((h2 + b2_ref[...]) * sc2 + sh2, 0.0)
    out_ref[...] = jnp.max(a2.reshape(bq, KNN, 128), axis=1)


def _run_passes(g3, z, g1, be1, w2, b2, g2, be2, n, BQ=400):
    grid = n // BQ
    n_edges = float(n * KNN)
    vec = pl.BlockSpec((1, 128), lambda i: (0, 0))
    g_spec = pl.BlockSpec((BQ, KNN, 128), lambda i: (i, 0, 0))
    z_spec = pl.BlockSpec((BQ, 128), lambda i: (i, 0))
    w_spec = pl.BlockSpec((128, 128), lambda i: (0, 0))
    acc = jax.ShapeDtypeStruct((1, 128), jnp.float32)

    s1, ss1 = pl.pallas_call(
        _stats1_body,
        grid=(grid,),
        in_specs=[g_spec, z_spec],
        out_specs=[vec, vec],
        out_shape=[acc, acc],
    )(g3, z)

    s2, ss2 = pl.pallas_call(
        functools.partial(_stats2_body, n_edges=n_edges),
        grid=(grid,),
        in_specs=[g_spec, z_spec, vec, vec, vec, vec, w_spec, vec],
        out_specs=[vec, vec],
        out_shape=[acc, acc],
    )(g3, z, s1, ss1, g1, be1, w2, b2)

    return pl.pallas_call(
        functools.partial(_final_body, n_edges=n_edges),
        grid=(grid,),
        in_specs=[g_spec, z_spec, vec, vec, vec, vec, w_spec, vec,
                  vec, vec, vec, vec],
        out_specs=pl.BlockSpec((BQ, 128), lambda i: (i, 0)),
        out_shape=jax.ShapeDtypeStruct((n, 128), jnp.float32),
    )(g3, z, s1, ss1, g1, be1, w2, b2, s2, ss2, g2, be2)


def kernel(p, x, b, W1, bias1, g1, be1, W2, bias2, g2, be2):
    n, c = x.shape
    bi = b.astype(jnp.int32)
    npad = ((n + 511) // 512) * 512
    pad = npad - n

    p_pad = jnp.pad(p, ((0, pad + 512), (0, 0)))
    b_pad = jnp.pad(bi, (0, pad + 512), constant_values=127)
    col = _knn(p_pad, b_pad[:, None], p_pad[:npad].T, b_pad[None, :npad],
               npad)[:, :n].T

    row_vec = lambda v: v[None, :]
    u, z = _uz(p, x, W1[:3], W1[3:], row_vec(bias1), n, c)

    e = n * KNN
    g = _sc_gather(u, col.reshape(e), e, 128)
    g3 = g.reshape(n, KNN, 128)

    x_agg = _run_passes(g3, z, row_vec(g1), row_vec(be1), W2,
                        row_vec(bias2), row_vec(g2), row_vec(be2), n)
    return (p, x_agg, b)
